# baseline (speedup 1.0000x reference)
import jax
import jax.numpy as jnp
from jax.experimental import pallas as pl

_K = 1024
_D = 64
_HW = 1024


def _rowsum_sq_lanes(m):
    sq = m * m
    acc = sq[:, 0:8]
    for j in range(1, 8):
        acc = acc + sq[:, 8 * j:8 * j + 8]
    t = acc[:, 0:4] + acc[:, 4:8]
    t = t[:, 0:2] + t[:, 2:4]
    return t[:, 0:1] + t[:, 1:2]


def _colsum_sq_sublanes(xb):
    sq = xb * xb

    def chunk(c):
        s = lambda i: sq[8 * c + i:8 * c + i + 1, :]
        return (((s(0) + s(1)) + (s(2) + s(3)))
                + ((s(4) + s(5)) + (s(6) + s(7))))

    acc = chunk(0)
    for c in range(1, 8):
        acc = acc + chunk(c)
    return acc


def _vq_body(x_ref, emb_ref, out_ref):
    xb = x_ref[0]
    emb = emb_ref[...]
    x2 = _colsum_sq_sublanes(xb)
    e2 = _rowsum_sq_lanes(emb)
    dot = jax.lax.dot_general(
        emb, xb, (((1,), (0,)), ((), ())),
        preferred_element_type=jnp.float32)
    d2 = (x2 + e2) - 2.0 * dot
    dist = jnp.sqrt(jnp.maximum(d2, 0.0))
    m = jnp.min(dist, axis=0, keepdims=True)
    ids = jax.lax.broadcasted_iota(jnp.int32, dist.shape, 0)
    cand = jnp.where(dist == m, ids, _K)
    out_ref[0] = jnp.min(cand, axis=0, keepdims=True)


def kernel(x, embeddings):
    B, d, H, W = x.shape
    x3 = x.reshape(B, d, H * W)
    out = pl.pallas_call(
        _vq_body,
        grid=(B,),
        in_specs=[
            pl.BlockSpec((1, d, H * W), lambda i: (i, 0, 0)),
            pl.BlockSpec((_K, d), lambda i: (0, 0)),
        ],
        out_specs=pl.BlockSpec((1, 1, H * W), lambda i: (i, 0, 0)),
        out_shape=jax.ShapeDtypeStruct((B, 1, H * W), jnp.int32),
    )(x3, embeddings)
    return out.reshape(B, H, W)


# speedup vs baseline: 1.8563x; 1.8563x over previous
import jax
import jax.numpy as jnp
from jax.experimental import pallas as pl
from jax.experimental.pallas import tpu as pltpu

_K = 1024
_D = 64
_HW = 1024


def _rowsum_sq_lanes(m):
    sq = m * m
    acc = sq[:, 0:8]
    for j in range(1, 8):
        acc = acc + sq[:, 8 * j:8 * j + 8]
    t = acc[:, 0:4] + acc[:, 4:8]
    t = t[:, 0:2] + t[:, 2:4]
    return t[:, 0:1] + t[:, 1:2]


def _colsum_sq_sublanes(xb):
    sq = xb * xb

    def chunk(c):
        s = lambda i: sq[8 * c + i:8 * c + i + 1, :]
        return (((s(0) + s(1)) + (s(2) + s(3)))
                + ((s(4) + s(5)) + (s(6) + s(7))))

    acc = chunk(0)
    for c in range(1, 8):
        acc = acc + chunk(c)
    return acc


def _succ(f):
    return jax.lax.bitcast_convert_type(
        jax.lax.bitcast_convert_type(f, jnp.int32) + 1, jnp.float32)


def _pred(f):
    return jax.lax.bitcast_convert_type(
        jax.lax.bitcast_convert_type(f, jnp.int32) - 1, jnp.float32)


def _vq_body(x_ref, emb_ref, out_ref, e2_ref, iota_ref):
    @pl.when(pl.program_id(0) == 0)
    def _init():
        e2_ref[...] = _rowsum_sq_lanes(emb_ref[...])
        iota_ref[...] = jax.lax.broadcasted_iota(jnp.int32, (_K, _HW), 0)

    xb = x_ref[0]
    x2 = _colsum_sq_sublanes(xb)
    dot = jax.lax.dot_general(
        emb_ref[...], xb, (((1,), (0,)), ((), ())),
        preferred_element_type=jnp.float32)
    b = (x2 + e2_ref[...]) - 2.0 * dot

    bmin = jnp.min(b, axis=0, keepdims=True)
    cmin = jnp.maximum(bmin, 0.0)
    m = jnp.sqrt(cmin)
    B = m * _succ(m)
    for _ in range(3):
        up = _succ(B)
        B = jnp.where(jnp.sqrt(up) == m, up, B)
    for _ in range(3):
        B = jnp.where(jnp.sqrt(B) == m, B, _pred(B))
    B = jnp.maximum(B, cmin)
    cand = jnp.where(b <= B, iota_ref[...], _K)
    out_ref[0] = jnp.min(cand, axis=0, keepdims=True)


def kernel(x, embeddings):
    B, d, H, W = x.shape
    x3 = x.reshape(B, d, H * W)
    out = pl.pallas_call(
        _vq_body,
        grid=(B,),
        in_specs=[
            pl.BlockSpec((1, d, H * W), lambda i: (i, 0, 0)),
            pl.BlockSpec((_K, d), lambda i: (0, 0)),
        ],
        out_specs=pl.BlockSpec((1, 1, H * W), lambda i: (i, 0, 0)),
        out_shape=jax.ShapeDtypeStruct((B, 1, H * W), jnp.int32),
        scratch_shapes=[
            pltpu.VMEM((_K, 1), jnp.float32),
            pltpu.VMEM((_K, _HW), jnp.int32),
        ],
    )(x3, embeddings)
    return out.reshape(B, H, W)
